# baseline (speedup 1.0000x reference)
import jax
import jax.numpy as jnp
from jax.experimental import pallas as pl
from jax.experimental.pallas import tpu as pltpu


def _erf(v):
    s = jnp.sign(v)
    av = jnp.abs(v)
    t = 1.0 / (1.0 + 0.3275911 * av)
    poly = t * (0.254829592 + t * (-0.284496736 + t * (1.421413741
           + t * (-1.453152027 + t * 1.061405429))))
    return s * (1.0 - poly * jnp.exp(-av * av))


def _gelu_exact(h):
    return 0.5 * h * (1.0 + _erf(h * 0.7071067811865476))


def _make_body(NCH, E, C, SB, S):
    def body(dm_ref, cm_ref, x_ref, w1_ref, b1_ref, w2_ref, b2_ref,
             out_ref, acc_scr, t_scr, w1_scr, w2_scr, y_scr, cm_bufs,
             sem1, sem2, csem):
        i = pl.program_id(0)

        @pl.when(i == 0)
        def _start_w():
            for e in range(E):
                pltpu.make_async_copy(w1_ref.at[e], w1_scr.at[e],
                                      sem1.at[e]).start()
                pltpu.make_async_copy(w2_ref.at[e], w2_scr.at[e],
                                      sem2.at[e]).start()

        @pl.when(i == NCH)
        def _start_cm():
            for q in (0, 1):
                pltpu.make_async_copy(
                    cm_ref.at[pl.ds(q * SB, SB), :, :], cm_bufs.at[q],
                    csem.at[q]).start()

        @pl.when(i < NCH)
        def _dispatch():
            t_scr[...] = jnp.swapaxes(dm_ref[...], 0, 1).astype(jnp.bfloat16)
            xb = x_ref[...].astype(jnp.bfloat16)
            for e in range(E):
                part = jax.lax.dot_general(
                    t_scr[e], xb,
                    dimension_numbers=(((0,), (0,)), ((), ())),
                    preferred_element_type=jnp.float32,
                )
                lo = e * C

                @pl.when(i == 0)
                def _(part=part, lo=lo):
                    acc_scr[lo:lo + C, :] = part

                @pl.when(i != 0)
                def _(part=part, lo=lo):
                    acc_scr[lo:lo + C, :] += part

        @pl.when((i >= NCH) & (i < NCH + E))
        def _ffn():
            e = i - NCH
            pltpu.make_async_copy(w1_ref.at[e], w1_scr.at[e], sem1.at[e]).wait()
            pltpu.make_async_copy(w2_ref.at[e], w2_scr.at[e], sem2.at[e]).wait()
            sl = pl.ds(e * C, C)
            de = acc_scr[sl, :]
            h = jnp.dot(de, w1_scr[e], preferred_element_type=jnp.float32)
            h = _gelu_exact(h + b1_ref[0])
            y = jnp.dot(h, w2_scr[e], preferred_element_type=jnp.float32)
            y_scr[sl, :] = (y + b2_ref[0]).astype(jnp.bfloat16)

        @pl.when(i >= NCH + E)
        def _combine():
            j = i - (NCH + E)
            pltpu.make_async_copy(
                cm_ref.at[pl.ds(j * SB, SB), :, :], cm_bufs.at[j % 2],
                csem.at[j]).wait()
            t_scr[...] = jnp.swapaxes(cm_bufs[j % 2], 0, 1).astype(jnp.bfloat16)

            @pl.when(j < NCH - 2)
            def _next_q():
                pltpu.make_async_copy(
                    cm_ref.at[pl.ds((j + 2) * SB, SB), :, :], cm_bufs.at[j % 2],
                    csem.at[j + 2]).start()
            acc = jnp.dot(t_scr[0], y_scr[0:C, :],
                          preferred_element_type=jnp.float32)
            for e in range(1, E):
                acc += jnp.dot(t_scr[e], y_scr[e * C:(e + 1) * C, :],
                               preferred_element_type=jnp.float32)
            out_ref[...] = acc

    return body


def kernel(x, dispatch_mask, combine_array, W1, b1, W2, b2):
    B, S, D = x.shape
    _, _, E, C = dispatch_mask.shape
    HE = W1.shape[2]
    EC = E * C

    Sb = 512
    NCH = S // Sb
    lastc = NCH - 1

    xs = x.reshape(S, D)
    dm = dispatch_mask.reshape(S, E, C)
    cm = combine_array.reshape(S, E, C)
    b1r = b1.reshape(E, 1, HE)
    b2r = b2.reshape(E, 1, D)

    grid = (NCH + E + NCH,)

    out = pl.pallas_call(
        _make_body(NCH, E, C, Sb, S),
        grid=grid,
        in_specs=[
            pl.BlockSpec((Sb, E, C), lambda i: (jnp.minimum(i, lastc), 0, 0)),
            pl.BlockSpec(memory_space=pltpu.MemorySpace.HBM),
            pl.BlockSpec((Sb, D), lambda i: (jnp.minimum(i, lastc), 0)),
            pl.BlockSpec(memory_space=pltpu.MemorySpace.HBM),
            pl.BlockSpec((1, 1, HE), lambda i: (jnp.clip(i - NCH, 0, E - 1), 0, 0)),
            pl.BlockSpec(memory_space=pltpu.MemorySpace.HBM),
            pl.BlockSpec((1, 1, D), lambda i: (jnp.clip(i - NCH, 0, E - 1), 0, 0)),
        ],
        out_specs=pl.BlockSpec((Sb, D), lambda i: (jnp.clip(i - (NCH + E), 0, lastc), 0)),
        out_shape=jax.ShapeDtypeStruct((S, D), jnp.float32),
        scratch_shapes=[
            pltpu.VMEM((EC, D), jnp.float32),
            pltpu.VMEM((E, Sb, C), jnp.bfloat16),
            pltpu.VMEM((E, D, HE), jnp.float32),
            pltpu.VMEM((E, HE, D), jnp.float32),
            pltpu.VMEM((EC, D), jnp.bfloat16),
            pltpu.VMEM((2, Sb, E, C), jnp.float32),
            pltpu.SemaphoreType.DMA((8,)),
            pltpu.SemaphoreType.DMA((8,)),
            pltpu.SemaphoreType.DMA((4,)),
        ],
    )(dm, cm, xs, W1, b1r, W2, b2r)
    return out.reshape(B, S, D)


# speedup vs baseline: 1.0235x; 1.0235x over previous
import jax
import jax.numpy as jnp
from jax.experimental import pallas as pl
from jax.experimental.pallas import tpu as pltpu


def _erf(v):
    s = jnp.sign(v)
    av = jnp.abs(v)
    t = 1.0 / (1.0 + 0.3275911 * av)
    poly = t * (0.254829592 + t * (-0.284496736 + t * (1.421413741
           + t * (-1.453152027 + t * 1.061405429))))
    return s * (1.0 - poly * jnp.exp(-av * av))


def _gelu_exact(h):
    return 0.5 * h * (1.0 + _erf(h * 0.7071067811865476))


def _make_body(NCH, E, C, SB, S):
    def body(dm_ref, cm_ref, x_ref, w1_ref, b1_ref, w2_ref, b2_ref,
             out_ref, acc_scr, t_scr, w1_scr, w2_scr, y_scr, cm_bufs,
             sem1, sem2, csem):
        i = pl.program_id(0)

        @pl.when(i == 0)
        def _start_w():
            for e in range(E):
                pltpu.make_async_copy(w1_ref.at[e], w1_scr.at[e],
                                      sem1.at[e]).start()
                pltpu.make_async_copy(w2_ref.at[e], w2_scr.at[e],
                                      sem2.at[e]).start()

        @pl.when(i == NCH - 1)
        def _start_cm():
            for q in (0, 1):
                pltpu.make_async_copy(
                    cm_ref.at[pl.ds(q * SB, SB), :, :], cm_bufs.at[q],
                    csem.at[q]).start()

        @pl.when(i < NCH)
        def _dispatch():
            t_scr[...] = jnp.swapaxes(dm_ref[...], 0, 1).astype(jnp.bfloat16)
            xb = x_ref[...].astype(jnp.bfloat16)
            for e in range(E):
                part = jax.lax.dot_general(
                    t_scr[e], xb,
                    dimension_numbers=(((0,), (0,)), ((), ())),
                    preferred_element_type=jnp.float32,
                )
                lo = e * C

                @pl.when(i == 0)
                def _(part=part, lo=lo):
                    acc_scr[lo:lo + C, :] = part

                @pl.when(i != 0)
                def _(part=part, lo=lo):
                    acc_scr[lo:lo + C, :] += part

        @pl.when((i >= NCH) & (i < NCH + E))
        def _ffn():
            e = i - NCH
            pltpu.make_async_copy(w1_ref.at[e], w1_scr.at[e], sem1.at[e]).wait()
            pltpu.make_async_copy(w2_ref.at[e], w2_scr.at[e], sem2.at[e]).wait()
            sl = pl.ds(e * C, C)
            de = acc_scr[sl, :]
            h = jnp.dot(de, w1_scr[e], preferred_element_type=jnp.float32)
            h = _gelu_exact(h + b1_ref[0])
            y = jnp.dot(h, w2_scr[e], preferred_element_type=jnp.float32)
            y_scr[sl, :] = (y + b2_ref[0]).astype(jnp.bfloat16)

        @pl.when(i >= NCH + E)
        def _combine():
            j = i - (NCH + E)
            pltpu.make_async_copy(
                cm_ref.at[pl.ds(j * SB, SB), :, :], cm_bufs.at[j % 2],
                csem.at[j]).wait()
            t_scr[...] = jnp.swapaxes(cm_bufs[j % 2], 0, 1).astype(jnp.bfloat16)

            @pl.when(j < NCH - 2)
            def _next_q():
                pltpu.make_async_copy(
                    cm_ref.at[pl.ds((j + 2) * SB, SB), :, :], cm_bufs.at[j % 2],
                    csem.at[j + 2]).start()
            acc = jnp.dot(t_scr[0], y_scr[0:C, :],
                          preferred_element_type=jnp.float32)
            for e in range(1, E):
                acc += jnp.dot(t_scr[e], y_scr[e * C:(e + 1) * C, :],
                               preferred_element_type=jnp.float32)
            out_ref[...] = acc

    return body


def kernel(x, dispatch_mask, combine_array, W1, b1, W2, b2):
    B, S, D = x.shape
    _, _, E, C = dispatch_mask.shape
    HE = W1.shape[2]
    EC = E * C

    Sb = 512
    NCH = S // Sb
    lastc = NCH - 1

    xs = x.reshape(S, D)
    dm = dispatch_mask.reshape(S, E, C)
    cm = combine_array.reshape(S, E, C)
    b1r = b1.reshape(E, 1, HE)
    b2r = b2.reshape(E, 1, D)

    grid = (NCH + E + NCH,)

    out = pl.pallas_call(
        _make_body(NCH, E, C, Sb, S),
        grid=grid,
        in_specs=[
            pl.BlockSpec((Sb, E, C), lambda i: (jnp.minimum(i, lastc), 0, 0)),
            pl.BlockSpec(memory_space=pltpu.MemorySpace.HBM),
            pl.BlockSpec((Sb, D), lambda i: (jnp.minimum(i, lastc), 0)),
            pl.BlockSpec(memory_space=pltpu.MemorySpace.HBM),
            pl.BlockSpec((1, 1, HE), lambda i: (jnp.clip(i - NCH, 0, E - 1), 0, 0)),
            pl.BlockSpec(memory_space=pltpu.MemorySpace.HBM),
            pl.BlockSpec((1, 1, D), lambda i: (jnp.clip(i - NCH, 0, E - 1), 0, 0)),
        ],
        out_specs=pl.BlockSpec((Sb, D), lambda i: (jnp.clip(i - (NCH + E), 0, lastc), 0)),
        out_shape=jax.ShapeDtypeStruct((S, D), jnp.float32),
        scratch_shapes=[
            pltpu.VMEM((EC, D), jnp.float32),
            pltpu.VMEM((E, Sb, C), jnp.bfloat16),
            pltpu.VMEM((E, D, HE), jnp.float32),
            pltpu.VMEM((E, HE, D), jnp.float32),
            pltpu.VMEM((EC, D), jnp.bfloat16),
            pltpu.VMEM((2, Sb, E, C), jnp.float32),
            pltpu.SemaphoreType.DMA((8,)),
            pltpu.SemaphoreType.DMA((8,)),
            pltpu.SemaphoreType.DMA((4,)),
        ],
    )(dm, cm, xs, W1, b1r, W2, b2r)
    return out.reshape(B, S, D)
